# baseline (speedup 1.0000x reference)
import jax
import jax.numpy as jnp
from jax import lax
from jax.experimental import pallas as pl
from jax.experimental.pallas import tpu as pltpu
from jax.experimental.pallas import tpu_sc as plsc

M = 524288
D = 64
B = 65536

NC = 2
NS = 16
NPASS = 32
RANGE = M // (NPASS * NC)
RSHIFT = 13
TROWS = RANGE // NS
SLICE = B // NS
CHUNKS = SLICE // 16
GARBAGE = RANGE
DUMPBIN = NPASS
LISTCAP = SLICE + 144
W = 128
WV = W // 16


def _body(mem_hbm, val_hbm, idx_hbm, out_hbm, idx_buf, lid_s, pos_s, fill,
          lid_win, pos_win, rows_buf, acc_a, acc_b, ss_a, ss_b, ws_a, ws_b,
          gsem):
    c = lax.axis_index("c")
    s = lax.axis_index("s")

    accs = (acc_a, acc_b)
    ssems = (ss_a, ss_b)
    wsems = (ws_a, ws_b)

    pltpu.sync_copy(idx_hbm.at[pl.ds(s * SLICE, SLICE)], idx_buf)

    lanes = lax.iota(jnp.int32, 16)
    ones = jnp.ones((16,), jnp.int32)
    zeros = jnp.zeros((16,), jnp.int32)

    def stage(p):
        base = (p * NC + c) * RANGE
        a = accs[p % 2]
        return pltpu.async_copy(
            mem_hbm.at[pl.ds(base + s * TROWS, TROWS)],
            a.at[pl.ds(s * TROWS, TROWS)], ssems[p % 2])

    stage_desc = {0: stage(0)}

    def bins_of(i):
        idxv = idx_buf[pl.ds(i * 16, 16)]
        r = idxv >> RSHIFT
        cm = (r & 1) ^ c
        pb = r >> 1
        return idxv, pb * (1 - cm) + DUMPBIN * cm

    fill[pl.ds(0, 16)] = zeros
    fill[pl.ds(16, 16)] = zeros
    fill[pl.ds(32, 16)] = zeros

    def count_chunk(i, _):
        _, binv = bins_of(i)
        plsc.addupdate_scatter(fill, [binv], ones)
        return 0

    lax.fori_loop(0, CHUNKS, count_chunk, 0)

    f0 = fill[pl.ds(0, 16)]
    e0 = plsc.cumsum(f0) - f0
    t0 = jnp.sum(f0)
    f1 = fill[pl.ds(16, 16)]
    e1 = t0 + plsc.cumsum(f1) - f1
    t1 = t0 + jnp.sum(f1)
    f2 = fill[pl.ds(32, 16)]
    e2 = t1 + plsc.cumsum(f2) - f2
    fill[pl.ds(0, 16)] = e0
    fill[pl.ds(16, 16)] = e1
    fill[pl.ds(32, 16)] = e2

    def scat_chunk(i, _):
        idxv, binv = bins_of(i)
        occ, _ = plsc.scan_count(binv)
        bf = plsc.load_gather(fill, [binv])
        off = bf + occ - 1
        plsc.store_scatter(lid_s, [off], idxv & (RANGE - 1))
        plsc.store_scatter(pos_s, [off], s * SLICE + i * 16 + lanes)
        plsc.addupdate_scatter(fill, [binv], ones)
        return 0

    lax.fori_loop(0, CHUNKS, scat_chunk, 0)
    fe0 = fill[pl.ds(0, 16)]
    fe1 = fill[pl.ds(16, 16)]

    def endof(b):
        return fe0[b] if b < 16 else fe1[b - 16]

    def adds(p, cur):
        end = endof(p)
        start = endof(p - 1) if p > 0 else jnp.int32(0)
        head = start & 7
        wstart = start - head
        total = head + (end - start)
        nch = (total + (W - 1)) // W

        def win(k, _):
            woff = pl.multiple_of(wstart + k * W, 8)
            for w in range(WV):
                g = k * W + w * 16 + lanes
                valid = (1 - (((g - head) >> 31) & 1)) * \
                        (((g - total) >> 31) & 1)
                lw = lid_s[pl.ds(woff + w * 16, 16)]
                pw = pos_s[pl.ds(woff + w * 16, 16)]
                lid_win[pl.ds(w * 16, 16)] = valid * lw + \
                    (1 - valid) * GARBAGE
                pos_win[pl.ds(w * 16, 16)] = valid * pw
            pltpu.async_copy(val_hbm.at[pos_win], rows_buf, gsem).wait()
            pltpu.sync_copy(rows_buf, cur.at[lid_win], add=True)
            return 0

        lax.fori_loop(0, nch, win, 0)

    wb_desc = {}
    for p in range(NPASS):
        cur = accs[p % 2]
        base = (p * NC + c) * RANGE
        if p + 1 < NPASS:
            if p >= 1:
                wb_desc[p - 1].wait()
            stage_desc[p + 1] = stage(p + 1)
        stage_desc[p].wait()
        plsc.subcore_barrier()
        adds(p, cur)
        plsc.subcore_barrier()
        wb_desc[p] = pltpu.async_copy(
            cur.at[pl.ds(s * TROWS, TROWS)],
            out_hbm.at[pl.ds(base + s * TROWS, TROWS)], wsems[p % 2])
    wb_desc[NPASS - 1].wait()


@jax.jit
def _scatter_add(mem, val, idx):
    mesh = plsc.VectorSubcoreMesh(core_axis_name="c", subcore_axis_name="s")
    return pl.kernel(
        _body,
        out_type=jax.ShapeDtypeStruct((M, D), jnp.float32),
        mesh=mesh,
        compiler_params=pltpu.CompilerParams(needs_layout_passes=False,
                                             use_tc_tiling_on_sc=False),
        scratch_types=[
            pltpu.VMEM((SLICE,), jnp.int32),
            pltpu.VMEM((LISTCAP,), jnp.int32),
            pltpu.VMEM((LISTCAP,), jnp.int32),
            pltpu.VMEM((48,), jnp.int32),
            pltpu.VMEM((W,), jnp.int32),
            pltpu.VMEM((W,), jnp.int32),
            pltpu.VMEM((W, D), jnp.float32),
            pltpu.VMEM_SHARED((RANGE + 8, D), jnp.float32),
            pltpu.VMEM_SHARED((RANGE + 8, D), jnp.float32),
            pltpu.SemaphoreType.DMA,
            pltpu.SemaphoreType.DMA,
            pltpu.SemaphoreType.DMA,
            pltpu.SemaphoreType.DMA,
            pltpu.SemaphoreType.DMA,
        ],
    )(mem, val, idx)


def kernel(mem, val, idx):
    return _scatter_add(mem, val, idx)
